# Initial kernel scaffold; baseline (speedup 1.0000x reference)
import jax
import jax.numpy as jnp
from jax.experimental import pallas as pl


def kernel(nfeat, efeat, We, be, Ws1, Wn1, b1, Ws2, Wn2, b2, edge_index):
    raise NotImplementedError("write your pallas kernel here")


# speedup vs baseline: 3.4357x; 3.4357x over previous
import functools

import jax
import jax.numpy as jnp
from jax import lax
from jax.experimental import pallas as pl
from jax.experimental.pallas import tpu as pltpu
from jax.experimental.pallas import tpu_sc as plsc

N = 10000
E = 320000
F = 128
EF = 16

NC = 2
NS = 16
NW = NC * NS
C = 128
CPW = 79
EPW = CPW * C
E_PAD = NW * EPW
ACC_ROWS = 10112
ZPT = ACC_ROWS // NS
RPT = 624
TAIL = N - NS * RPT
DW = 16


def _zero_vec():
    return jnp.zeros((16,), jnp.float32)


def _sc_degree(dst):
    mesh = plsc.VectorSubcoreMesh(core_axis_name="c", subcore_axis_name="s")

    @functools.partial(
        pl.kernel,
        out_type=jax.ShapeDtypeStruct((NC, N, F), jnp.float32),
        mesh=mesh,
        scratch_types=[
            pltpu.VMEM((C,), jnp.int32),
            pltpu.VMEM((C, F), jnp.float32),
            pltpu.VMEM_SHARED((ACC_ROWS, F), jnp.float32),
        ],
    )
    def kern(dst_hbm, out_deg, dstv, ones, accd):
        cid = lax.axis_index("c")
        sid = lax.axis_index("s")
        wid = sid * NC + cid
        zero = _zero_vec()

        def zrow(r, carry):
            for j in range(F // 16):
                ones[r, pl.ds(j * 16, 16)] = zero
            return carry
        lax.fori_loop(0, C, zrow, 0)
        for i in range((ZPT + C - 1) // C):
            nr = min(C, ZPT - i * C)
            r0 = sid * ZPT + i * C
            pltpu.sync_copy(ones.at[pl.ds(0, nr)], accd.at[pl.ds(r0, nr)])

        one = jnp.ones((16,), jnp.float32)

        def initrow(r, carry):
            for j in range(F // 16):
                ones[r, pl.ds(j * 16, 16)] = one
            return carry
        lax.fori_loop(0, C, initrow, 0)
        plsc.subcore_barrier()

        base = wid * EPW

        def chunk(k, carry):
            off = base + k * C
            pltpu.sync_copy(dst_hbm.at[pl.ds(off, C)], dstv)
            pltpu.sync_copy(ones, accd.at[dstv], add=True)
            return carry
        lax.fori_loop(0, CPW, chunk, 0)
        plsc.subcore_barrier()

        def cpout(row, nr):
            pltpu.sync_copy(accd.at[pl.ds(row, nr)], ones.at[pl.ds(0, nr)])
            pltpu.sync_copy(ones.at[pl.ds(0, nr)], out_deg.at[cid, pl.ds(row, nr)])

        for i in range((RPT + C - 1) // C):
            cpout(sid * RPT + i * C, min(C, RPT - i * C))

        @pl.when(sid == NS - 1)
        def _tail():
            cpout(NS * RPT, TAIL)

    return kern(dst)


def _sc_edge_pass_a(nfeat, e, src, dst):
    mesh = plsc.VectorSubcoreMesh(core_axis_name="c", subcore_axis_name="s")

    @functools.partial(
        pl.kernel,
        out_type=jax.ShapeDtypeStruct((NC, N, F), jnp.float32),
        mesh=mesh,
        scratch_types=[
            pltpu.VMEM((C,), jnp.int32),
            pltpu.VMEM((C,), jnp.int32),
            pltpu.VMEM((C, F), jnp.float32),
            pltpu.VMEM((C, F), jnp.float32),
            pltpu.VMEM_SHARED((ACC_ROWS, F), jnp.float32),
            pltpu.SemaphoreType.DMA,
        ],
    )
    def kern(nfeat_hbm, e_hbm, src_hbm, dst_hbm, out_sum,
             srcv, dstv, rows, ev, acc, sem):
        cid = lax.axis_index("c")
        sid = lax.axis_index("s")
        wid = sid * NC + cid
        zero = _zero_vec()

        def initrow(r, carry):
            for j in range(F // 16):
                rows[r, pl.ds(j * 16, 16)] = zero
            return carry
        lax.fori_loop(0, C, initrow, 0)

        for i in range((ZPT + C - 1) // C):
            nr = min(C, ZPT - i * C)
            r0 = sid * ZPT + i * C
            pltpu.sync_copy(rows.at[pl.ds(0, nr)], acc.at[pl.ds(r0, nr)])
        plsc.subcore_barrier()

        base = wid * EPW

        def chunk(k, carry):
            off = base + k * C
            pltpu.sync_copy(src_hbm.at[pl.ds(off, C)], srcv)
            pltpu.sync_copy(dst_hbm.at[pl.ds(off, C)], dstv)
            pltpu.async_copy(nfeat_hbm.at[srcv], rows, sem).wait()
            pltpu.sync_copy(e_hbm.at[pl.ds(off, C)], ev)

            def mrow(r, c2):
                for j in range(F // 16):
                    s = pl.ds(j * 16, 16)
                    rows[r, s] = rows[r, s] * ev[r, s]
                return c2
            lax.fori_loop(0, C, mrow, 0)

            pltpu.sync_copy(rows, acc.at[dstv], add=True)
            return carry
        lax.fori_loop(0, CPW, chunk, 0)
        plsc.subcore_barrier()

        def cpout(row, nr):
            pltpu.sync_copy(acc.at[pl.ds(row, nr)], rows.at[pl.ds(0, nr)])
            pltpu.sync_copy(rows.at[pl.ds(0, nr)], out_sum.at[cid, pl.ds(row, nr)])

        for i in range((RPT + C - 1) // C):
            cpout(sid * RPT + i * C, min(C, RPT - i * C))

        @pl.when(sid == NS - 1)
        def _tail():
            cpout(NS * RPT, TAIL)

    return kern(nfeat, e, src, dst)


def _sc_gather_scatter(table, src, dst):
    mesh = plsc.VectorSubcoreMesh(core_axis_name="c", subcore_axis_name="s")

    @functools.partial(
        pl.kernel,
        out_type=jax.ShapeDtypeStruct((NC, N, F), jnp.float32),
        mesh=mesh,
        scratch_types=[
            pltpu.VMEM((C,), jnp.int32),
            pltpu.VMEM((C,), jnp.int32),
            pltpu.VMEM((C, F), jnp.float32),
            pltpu.VMEM_SHARED((ACC_ROWS, F), jnp.float32),
            pltpu.SemaphoreType.DMA,
        ],
    )
    def kern(table_hbm, src_hbm, dst_hbm, out_sum, srcv, dstv, rows, acc, sem):
        cid = lax.axis_index("c")
        sid = lax.axis_index("s")
        wid = sid * NC + cid
        zero = _zero_vec()

        def initrow(r, carry):
            for j in range(F // 16):
                rows[r, pl.ds(j * 16, 16)] = zero
            return carry
        lax.fori_loop(0, C, initrow, 0)
        for i in range((ZPT + C - 1) // C):
            nr = min(C, ZPT - i * C)
            r0 = sid * ZPT + i * C
            pltpu.sync_copy(rows.at[pl.ds(0, nr)], acc.at[pl.ds(r0, nr)])
        plsc.subcore_barrier()

        base = wid * EPW

        def chunk(k, carry):
            off = base + k * C
            pltpu.sync_copy(src_hbm.at[pl.ds(off, C)], srcv)
            pltpu.sync_copy(dst_hbm.at[pl.ds(off, C)], dstv)
            pltpu.async_copy(table_hbm.at[srcv], rows, sem).wait()
            pltpu.sync_copy(rows, acc.at[dstv], add=True)
            return carry
        lax.fori_loop(0, CPW, chunk, 0)
        plsc.subcore_barrier()

        def cpout(row, nr):
            pltpu.sync_copy(acc.at[pl.ds(row, nr)], rows.at[pl.ds(0, nr)])
            pltpu.sync_copy(rows.at[pl.ds(0, nr)], out_sum.at[cid, pl.ds(row, nr)])

        for i in range((RPT + C - 1) // C):
            cpout(sid * RPT + i * C, min(C, RPT - i * C))

        @pl.when(sid == NS - 1)
        def _tail():
            cpout(NS * RPT, TAIL)

    return kern(table, src, dst)


def _tc_edge_encoder(efeat_pad, WeT, be2):
    BE = 2048
    grid = (E_PAD // BE,)

    def body(ef, w, b, o):
        o[...] = jax.nn.relu(
            jnp.dot(ef[...], w[...], preferred_element_type=jnp.float32) + b[...])

    return pl.pallas_call(
        body,
        grid=grid,
        in_specs=[
            pl.BlockSpec((BE, EF), lambda i: (i, 0)),
            pl.BlockSpec((EF, F), lambda i: (0, 0)),
            pl.BlockSpec((1, F), lambda i: (0, 0)),
        ],
        out_specs=pl.BlockSpec((BE, F), lambda i: (i, 0)),
        out_shape=jax.ShapeDtypeStruct((E_PAD, F), jnp.float32),
    )(efeat_pad, WeT, be2)


def _tc_layer1(nfeat, s0, s1, d0, d1, WsaT, WsbT, WnaT, WnbT, b12):
    BN = 2000
    grid = (N // BN,)

    def body(nf, a0, a1, e0, e1, wsa, wsb, wna, wnb, b, z1, g1, dinv):
        deg = e0[...][:, :1] + e1[...][:, :1]
        degc = jnp.maximum(deg, 1.0)
        hn = (a0[...] + a1[...]) / degc
        nfb = nf[...]
        z1[...] = (jnp.dot(nfb, wsa[...], preferred_element_type=jnp.float32)
                   + jnp.dot(hn, wsb[...], preferred_element_type=jnp.float32)
                   + b[...])
        g1[...] = (jnp.dot(nfb, wna[...], preferred_element_type=jnp.float32)
                   + jnp.dot(hn, wnb[...], preferred_element_type=jnp.float32))
        dinv[...] = jnp.broadcast_to(1.0 / degc, (BN, DW))

    row = pl.BlockSpec((BN, F), lambda i: (i, 0))
    degs = pl.BlockSpec((BN, DW), lambda i: (i, 0))
    wspec = pl.BlockSpec((F, F), lambda i: (0, 0))
    return pl.pallas_call(
        body,
        grid=grid,
        in_specs=[row, row, row, row, row, wspec, wspec, wspec, wspec,
                  pl.BlockSpec((1, F), lambda i: (0, 0))],
        out_specs=[row, row, degs],
        out_shape=[
            jax.ShapeDtypeStruct((N, F), jnp.float32),
            jax.ShapeDtypeStruct((N, F), jnp.float32),
            jax.ShapeDtypeStruct((N, DW), jnp.float32),
        ],
    )(nfeat, s0, s1, d0, d1, WsaT, WsbT, WnaT, WnbT, b12)


def _tc_layer2(z1, q0, q1, dinv, Ws2T, Wn2T, b22):
    BN = 2000
    grid = (N // BN,)

    def body(z, a0, a1, di, ws, wn, b, z2, g2):
        h1 = jax.nn.relu(z[...] + (a0[...] + a1[...]) * di[...][:, :1])
        z2[...] = (jnp.dot(h1, ws[...], preferred_element_type=jnp.float32)
                   + b[...])
        g2[...] = jnp.dot(h1, wn[...], preferred_element_type=jnp.float32)

    row = pl.BlockSpec((BN, F), lambda i: (i, 0))
    degs = pl.BlockSpec((BN, DW), lambda i: (i, 0))
    wspec = pl.BlockSpec((F, F), lambda i: (0, 0))
    return pl.pallas_call(
        body,
        grid=grid,
        in_specs=[row, row, row, degs, wspec, wspec,
                  pl.BlockSpec((1, F), lambda i: (0, 0))],
        out_specs=[row, row],
        out_shape=[
            jax.ShapeDtypeStruct((N, F), jnp.float32),
            jax.ShapeDtypeStruct((N, F), jnp.float32),
        ],
    )(z1, q0, q1, dinv, Ws2T, Wn2T, b22)


def _tc_final(z2, r0, r1, dinv):
    BN = 2000
    grid = (N // BN,)

    def body(z, a0, a1, di, o):
        o[...] = z[...] + (a0[...] + a1[...]) * di[...][:, :1]

    row = pl.BlockSpec((BN, F), lambda i: (i, 0))
    degs = pl.BlockSpec((BN, DW), lambda i: (i, 0))
    return pl.pallas_call(
        body,
        grid=grid,
        in_specs=[row, row, row, degs],
        out_specs=row,
        out_shape=jax.ShapeDtypeStruct((N, F), jnp.float32),
    )(z2, r0, r1, dinv)


def kernel(nfeat, efeat, We, be, Ws1, Wn1, b1, Ws2, Wn2, b2, edge_index):
    pad = E_PAD - E
    src = jnp.concatenate([edge_index[0], jnp.zeros((pad,), jnp.int32)])
    dst = jnp.concatenate([edge_index[1], jnp.full((pad,), N, jnp.int32)])
    efp = jnp.concatenate([efeat, jnp.zeros((pad, EF), jnp.float32)], axis=0)

    e = _tc_edge_encoder(efp, We.T, be.reshape(1, F))
    degp = _sc_degree(dst)
    sump = _sc_edge_pass_a(nfeat, e, src, dst)
    z1, g1, dinv = _tc_layer1(
        nfeat, sump[0], sump[1], degp[0], degp[1],
        Ws1[:, :F].T, Ws1[:, F:].T, Wn1[:, :F].T, Wn1[:, F:].T,
        b1.reshape(1, F))
    qp = _sc_gather_scatter(g1, src, dst)
    z2, g2 = _tc_layer2(z1, qp[0], qp[1], dinv, Ws2.T, Wn2.T, b2.reshape(1, F))
    rp = _sc_gather_scatter(g2, src, dst)
    return _tc_final(z2, rp[0], rp[1], dinv)
